# baseline (speedup 1.0000x reference)
import jax
import jax.numpy as jnp
from jax import lax
from jax.experimental import pallas as pl
from jax.experimental.pallas import tpu as pltpu
from jax.experimental.pallas import tpu_sc as plsc

HS = 20
D = 256
H = 8
DH = D // H
N = 10000
E = 160000
EH = E // 2

NC = 2
NS = 16
NW = NC * NS

CH = 128
BE = 1000
BN = 2000
NPT = N // NS
SLAB = 632
SLAB_LAST = N - 15 * SLAB

_mesh = plsc.VectorSubcoreMesh(core_axis_name="c", subcore_axis_name="s")
_SC_UNTILED = pltpu.CompilerParams(use_tc_tiling_on_sc=False)


def _make_gather_feats(ecnt):
    nchunk = ecnt // CH

    def body_fn(tbl, src, dst, fs_out, fd_out,
                idx_s, idx_d, rows, gs0, gs1, ws0, ws1):
        wid = lax.axis_index("s") * NC + lax.axis_index("c")
        iters = (nchunk + NW - 1) // NW
        outs = (fs_out, fd_out)
        gsems = (gs0, gs1)
        wsems = (ws0, ws1)
        idxs = (idx_s, idx_d)

        def body(j, _):
            cid = j * NW + wid

            @pl.when(cid < nchunk)
            def _():
                base = cid * CH

                @pl.when(j > 0)
                def _():
                    for i in range(2):
                        pltpu.make_async_copy(
                            rows.at[i], outs[i].at[pl.ds(base, CH)], wsems[i]
                        ).wait()

                pltpu.sync_copy(src.at[pl.ds(base, CH)], idx_s)
                pltpu.sync_copy(dst.at[pl.ds(base, CH)], idx_d)
                for i in range(2):
                    pltpu.async_copy(tbl.at[idxs[i]], rows.at[i], gsems[i])
                for i in range(2):
                    pltpu.make_async_copy(tbl.at[idxs[i]], rows.at[i],
                                          gsems[i]).wait()
                    pltpu.async_copy(rows.at[i], outs[i].at[pl.ds(base, CH)],
                                     wsems[i])

            return 0

        lax.fori_loop(0, iters, body, 0)
        for i in range(2):
            pltpu.make_async_copy(rows.at[i], outs[i].at[pl.ds(0, CH)],
                                  wsems[i]).wait()

    return pl.kernel(
        body_fn,
        out_type=[
            jax.ShapeDtypeStruct((ecnt, 128), jnp.float32),
            jax.ShapeDtypeStruct((ecnt, 128), jnp.float32),
        ],
        mesh=_mesh,
        scratch_types=[
            pltpu.VMEM((CH,), jnp.int32),
            pltpu.VMEM((CH,), jnp.int32),
            pltpu.VMEM((2, CH, 128), jnp.float32),
            pltpu.SemaphoreType.DMA,
            pltpu.SemaphoreType.DMA,
            pltpu.SemaphoreType.DMA,
            pltpu.SemaphoreType.DMA,
        ],
    )


def _make_gather_qkv(ecnt):
    nchunk = ecnt // CH

    def body_fn(q, kn, vn, src, dst, qd_out, ks_out, vs_out,
                idx_s, idx_d, rows, gs0, gs1, gs2, ws0, ws1, ws2):
        wid = lax.axis_index("s") * NC + lax.axis_index("c")
        iters = (nchunk + NW - 1) // NW
        outs = (qd_out, ks_out, vs_out)
        gsems = (gs0, gs1, gs2)
        wsems = (ws0, ws1, ws2)
        tbls = (q, kn, vn)

        def body(j, _):
            cid = j * NW + wid

            @pl.when(cid < nchunk)
            def _():
                base = cid * CH

                @pl.when(j > 0)
                def _():
                    for i in range(3):
                        pltpu.make_async_copy(
                            rows.at[i], outs[i].at[pl.ds(base, CH)], wsems[i]
                        ).wait()

                pltpu.sync_copy(src.at[pl.ds(base, CH)], idx_s)
                pltpu.sync_copy(dst.at[pl.ds(base, CH)], idx_d)
                idxs = (idx_d, idx_s, idx_s)
                for i in range(3):
                    pltpu.async_copy(tbls[i].at[idxs[i]], rows.at[i], gsems[i])
                for i in range(3):
                    pltpu.make_async_copy(tbls[i].at[idxs[i]], rows.at[i],
                                          gsems[i]).wait()
                    pltpu.async_copy(rows.at[i], outs[i].at[pl.ds(base, CH)],
                                     wsems[i])

            return 0

        lax.fori_loop(0, iters, body, 0)
        for i in range(3):
            pltpu.make_async_copy(rows.at[i], outs[i].at[pl.ds(0, CH)],
                                  wsems[i]).wait()

    return pl.kernel(
        body_fn,
        out_type=[
            jax.ShapeDtypeStruct((ecnt, 128), jnp.int32),
            jax.ShapeDtypeStruct((ecnt, 128), jnp.int32),
            jax.ShapeDtypeStruct((ecnt, 128), jnp.int32),
        ],
        mesh=_mesh,
        scratch_types=[
            pltpu.VMEM((CH,), jnp.int32),
            pltpu.VMEM((CH,), jnp.int32),
            pltpu.VMEM((3, CH, 128), jnp.int32),
            pltpu.SemaphoreType.DMA,
            pltpu.SemaphoreType.DMA,
            pltpu.SemaphoreType.DMA,
            pltpu.SemaphoreType.DMA,
            pltpu.SemaphoreType.DMA,
            pltpu.SemaphoreType.DMA,
        ],
    )


def _make_scatter(ecnt):
    nchunk = ecnt // CH

    def body_fn(dst, msg, zn, aggr_out, shared, idx_b, mrows, ms0, ms1, lsem):
        c = lax.axis_index("c")
        s = lax.axis_index("s")
        msems = (ms0, ms1)

        @pl.when(s < 15)
        def _():
            pltpu.sync_copy(zn.at[pl.ds(s * SLAB, SLAB)],
                            shared.at[pl.ds(s * SLAB, SLAB)])

        @pl.when(s == 15)
        def _():
            pltpu.sync_copy(zn.at[pl.ds(15 * SLAB, SLAB_LAST)],
                            shared.at[pl.ds(15 * SLAB, SLAB_LAST)])

        plsc.subcore_barrier()

        iters = (nchunk + NS - 1) // NS

        def body(jj, _):
            for p in range(2):
                j = jj * 2 + p
                cid = j * NS + s

                @pl.when(cid < nchunk)
                def _():
                    base = cid * CH

                    @pl.when(j > 1)
                    def _():
                        pltpu.make_async_copy(
                            mrows.at[p], shared.at[idx_b.at[p]],
                            msems[p]).wait()

                    pltpu.sync_copy(dst.at[pl.ds(base, CH)], idx_b.at[p])
                    pltpu.async_copy(msg.at[c, pl.ds(base, CH)], mrows.at[p],
                                     lsem)
                    pltpu.make_async_copy(msg.at[c, pl.ds(base, CH)],
                                          mrows.at[p], lsem).wait()
                    pltpu.async_copy(mrows.at[p], shared.at[idx_b.at[p]],
                                     msems[p], add=True)

            return 0

        lax.fori_loop(0, (iters + 1) // 2, body, 0)
        for p in range(2):
            pltpu.make_async_copy(mrows.at[p], shared.at[idx_b.at[p]],
                                  msems[p]).wait()

        plsc.subcore_barrier()

        @pl.when(s < 15)
        def _():
            pltpu.sync_copy(shared.at[pl.ds(s * SLAB, SLAB)],
                            aggr_out.at[c, pl.ds(s * SLAB, SLAB)])

        @pl.when(s == 15)
        def _():
            pltpu.sync_copy(shared.at[pl.ds(15 * SLAB, SLAB_LAST)],
                            aggr_out.at[c, pl.ds(15 * SLAB, SLAB_LAST)])

    return pl.kernel(
        body_fn,
        out_type=jax.ShapeDtypeStruct((NC, N, 128), jnp.float32),
        mesh=_mesh,
        scratch_types=[
            pltpu.VMEM_SHARED((N, 128), jnp.float32),
            pltpu.VMEM((2, CH), jnp.int32),
            pltpu.VMEM((2, CH, 128), jnp.float32),
            pltpu.SemaphoreType.DMA,
            pltpu.SemaphoreType.DMA,
            pltpu.SemaphoreType.DMA,
        ],
    )


def _make_scatter_den(ecnt):
    nchunk = ecnt // CH

    def body_fn(dst, ex, zd, den_out, shared_den, idx_b, erows,
                ds0, ds1, lsem):
        c = lax.axis_index("c")
        s = lax.axis_index("s")
        wid = s * NC + c
        dsems = (ds0, ds1)

        pltpu.sync_copy(zd.at[pl.ds(s * NPT, NPT)],
                        shared_den.at[pl.ds(s * NPT, NPT)])
        plsc.subcore_barrier()

        iters = (nchunk + NW - 1) // NW

        def body(jj, _):
            for p in range(2):
                j = jj * 2 + p
                cid = j * NW + wid

                @pl.when(cid < nchunk)
                def _():
                    base = cid * CH

                    @pl.when(j > 1)
                    def _():
                        pltpu.make_async_copy(
                            erows.at[p], shared_den.at[idx_b.at[p]],
                            dsems[p]).wait()

                    pltpu.sync_copy(dst.at[pl.ds(base, CH)], idx_b.at[p])
                    exs = ex.at[pl.ds(base, CH), pl.ds(0, 16)]
                    pltpu.async_copy(exs, erows.at[p], lsem)
                    pltpu.make_async_copy(exs, erows.at[p], lsem).wait()
                    pltpu.async_copy(erows.at[p], shared_den.at[idx_b.at[p]],
                                     dsems[p], add=True)

            return 0

        lax.fori_loop(0, (iters + 1) // 2, body, 0)
        for p in range(2):
            pltpu.make_async_copy(erows.at[p], shared_den.at[idx_b.at[p]],
                                  dsems[p]).wait()

        plsc.subcore_barrier()
        pltpu.sync_copy(shared_den.at[pl.ds(s * NPT, NPT)],
                        den_out.at[c, pl.ds(s * NPT, NPT)])

    return pl.kernel(
        body_fn,
        out_type=jax.ShapeDtypeStruct((NC, N, 16), jnp.float32),
        mesh=_mesh,
        scratch_types=[
            pltpu.VMEM_SHARED((N, 16), jnp.float32),
            pltpu.VMEM((2, CH), jnp.int32),
            pltpu.VMEM((2, CH, 16), jnp.float32),
            pltpu.SemaphoreType.DMA,
            pltpu.SemaphoreType.DMA,
            pltpu.SemaphoreType.DMA,
        ],
        compiler_params=_SC_UNTILED,
    )


_gather_feats_h = _make_gather_feats(EH)
_gather_qkv_h = _make_gather_qkv(EH)
_scatter_h = _make_scatter(EH)
_scatter_den_h = _make_scatter_den(EH)


def _ln_in(x, g, b):
    m = jnp.mean(x, axis=-1, keepdims=True)
    v = jnp.mean((x - m) ** 2, axis=-1, keepdims=True)
    return (x - m) * lax.rsqrt(v + 1e-5) * g + b


def _dot(a, b):
    return jnp.dot(a, b, preferred_element_type=jnp.float32)


def _pack_bf16(even, odd):
    ue = lax.bitcast_convert_type(even, jnp.uint32)
    ue = ue + jnp.uint32(0x7FFF) + ((ue >> jnp.uint32(16)) & jnp.uint32(1))
    uo = lax.bitcast_convert_type(odd, jnp.uint32)
    uo = uo + jnp.uint32(0x7FFF) + ((uo >> jnp.uint32(16)) & jnp.uint32(1))
    packed = (uo & jnp.uint32(0xFFFF0000)) | (ue >> jnp.uint32(16))
    return lax.bitcast_convert_type(packed, jnp.int32)


def _unpack_bf16(xi):
    u = lax.bitcast_convert_type(xi, jnp.uint32)
    even = lax.bitcast_convert_type(u << jnp.uint32(16), jnp.float32)
    odd = lax.bitcast_convert_type(u & jnp.uint32(0xFFFF0000), jnp.float32)
    return even, odd


def _enc_body(fs, fd, sa_m, sa2_m, sb_m, we128, b1e, ge, be_, w2e, b2e,
              wr128, b1r, gr, br, w2r, b2r,
              ga1, ba1, wa, ba, ga2, ba2, ee_out):
    fs_ = fs[...]
    fd_ = fd[...]
    lft = _dot(fs_, sa_m[...]) + _dot(fd_, sa2_m[...])
    rgt = _dot(fd_, sb_m[...])
    prods = lft * rgt
    e = _dot(prods, we128[...]) + b1e[...]
    e = jnp.maximum(_ln_in(e, ge[...], be_[...]), 0.0).astype(jnp.bfloat16)
    e = _dot(e, w2e[...]) + b2e[...]
    r = _dot(prods, wr128[...]) + b1r[...]
    r = jnp.maximum(_ln_in(r, gr[...], br[...]), 0.0).astype(jnp.bfloat16)
    r = _dot(r, w2r[...]) + b2r[...]
    ee = e + r
    ee = jnp.maximum(_ln_in(ee, ga1[...], ba1[...]), 0.0).astype(jnp.bfloat16)
    ee = _dot(ee, wa[...]) + ba[...]
    ee_out[...] = _ln_in(ee, ga2[...], ba2[...]).astype(jnp.bfloat16)


def _prep_body(x, wq_e, wq_o, bq_e, bq_o, wk_e, wk_o, bk_e, bk_o,
               wv_e, wv_o, bv_e, bv_o, g1, b1,
               xn_out, q_out, k_out, v_out):
    xn = _ln_in(x[...], g1[...], b1[...])
    xn_out[...] = xn
    q_out[...] = _pack_bf16(_dot(xn, wq_e[...]) + bq_e[...],
                            _dot(xn, wq_o[...]) + bq_o[...])
    k_out[...] = _pack_bf16(_dot(xn, wk_e[...]) + bk_e[...],
                            _dot(xn, wk_o[...]) + bk_o[...])
    v_out[...] = _pack_bf16(_dot(xn, wv_e[...]) + bv_e[...],
                            _dot(xn, wv_o[...]) + bv_o[...])


def _att_body(qd, ks, vs, ee, wke_e, wke_o, bke_e, bke_o,
              wve_e, wve_o, bve_e, bve_o, m16, c16,
              msg_out, ex_out):
    ee_ = ee[...]
    qde, qdo = _unpack_bf16(qd[...])
    kse, kso = _unpack_bf16(ks[...])
    vse, vso = _unpack_bf16(vs[...])
    ke_e = _dot(ee_, wke_e[...]) + bke_e[...]
    ke_o = _dot(ee_, wke_o[...]) + bke_o[...]
    prod = qde * (kse + ke_e) + qdo * (kso + ke_o)
    s = _dot(prod, m16[...])
    ex = jnp.exp(s * (1.0 / (DH ** 0.5)))
    ve_e = _dot(ee_, wve_e[...]) + bve_e[...]
    ve_o = _dot(ee_, wve_o[...]) + bve_o[...]
    msg_out[0] = (vse + ve_e) * ex
    msg_out[1] = (vso + ve_o) * ex
    ex_out[...] = _dot(ex, c16[...])


def _upd_body(aggr_a, aggr_b, den_a, den_b, xn, x_in, st128,
              wih_ee, wih_oe, wih_eo, wih_oo, whh_e, whh_o, bg_e, bg_o,
              wsf_e, wsf_o, bsf_e, bsf_o, wo_e, wo_o, bo,
              g2, b2, wm1, bm1, wm2, bm2, gf, bf, out):
    den_e = _dot(den_a[0] + den_a[1] + den_b[0] + den_b[1], st128[...])
    rcp = 1.0 / (den_e + 1e-16)
    a_e = (aggr_a[0] + aggr_b[0]) * rcp
    a_o = (aggr_a[1] + aggr_b[1]) * rcp
    xn_ = xn[...]
    gate_e = jax.nn.sigmoid(_dot(a_e, wih_ee[...]) + _dot(a_o, wih_oe[...])
                            + _dot(xn_, whh_e[...]) + bg_e[...])
    gate_o = jax.nn.sigmoid(_dot(a_e, wih_eo[...]) + _dot(a_o, wih_oo[...])
                            + _dot(xn_, whh_o[...]) + bg_o[...])
    sf_e = _dot(xn_, wsf_e[...]) + bsf_e[...]
    sf_o = _dot(xn_, wsf_o[...]) + bsf_o[...]
    upd_e = a_e + gate_e * (sf_e - a_e)
    upd_o = a_o + gate_o * (sf_o - a_o)
    x = x_in[...] + _dot(upd_e, wo_e[...]) + _dot(upd_o, wo_o[...]) + bo[...]
    xn2 = _ln_in(x, g2[...], b2[...])
    h = jnp.maximum(_dot(xn2, wm1[...]) + bm1[...], 0.0)
    h = _dot(h, wm2[...]) + bm2[...]
    out[...] = _ln_in(x + h, gf[...], bf[...])


def _full(shape):
    nd = len(shape)
    return pl.BlockSpec(shape, lambda i: (0,) * nd)


_TC_PARAMS = pltpu.CompilerParams(
    dimension_semantics=("arbitrary",),
)


def _call_enc(fs, fd, *ws):
    ecnt = fs.shape[0]
    grid = (ecnt // BE,)
    especs = [pl.BlockSpec((BE, 128), lambda i: (i, 0))] * 2
    wspecs = [_full(w.shape) for w in ws]
    return pl.pallas_call(
        _enc_body,
        grid=grid,
        in_specs=especs + wspecs,
        out_specs=pl.BlockSpec((BE, D), lambda i: (i, 0)),
        out_shape=jax.ShapeDtypeStruct((ecnt, D), jnp.bfloat16),
        compiler_params=_TC_PARAMS,
    )(fs, fd, *ws)


def _call_prep(x, *ws):
    grid = (N // BN,)
    wspecs = [_full(w.shape) for w in ws]
    blk = pl.BlockSpec((BN, D), lambda i: (i, 0))
    hblk = pl.BlockSpec((BN, 128), lambda i: (i, 0))
    return pl.pallas_call(
        _prep_body,
        grid=grid,
        in_specs=[blk] + wspecs,
        out_specs=[blk] + [hblk] * 3,
        out_shape=[jax.ShapeDtypeStruct((N, D), jnp.float32)]
        + [jax.ShapeDtypeStruct((N, 128), jnp.int32)] * 3,
        compiler_params=_TC_PARAMS,
    )(x, *ws)


def _call_att(qd, ks, vs, ee, *ws):
    ecnt = qd.shape[0]
    grid = (ecnt // BE,)
    hblk = pl.BlockSpec((BE, 128), lambda i: (i, 0))
    eblk = pl.BlockSpec((BE, D), lambda i: (i, 0))
    wspecs = [_full(w.shape) for w in ws]
    return pl.pallas_call(
        _att_body,
        grid=grid,
        in_specs=[hblk] * 3 + [eblk] + wspecs,
        out_specs=[
            pl.BlockSpec((2, BE, 128), lambda i: (0, i, 0)),
            pl.BlockSpec((BE, 128), lambda i: (i, 0)),
        ],
        out_shape=[
            jax.ShapeDtypeStruct((2, ecnt, 128), jnp.float32),
            jax.ShapeDtypeStruct((ecnt, 128), jnp.float32),
        ],
        compiler_params=_TC_PARAMS,
    )(qd, ks, vs, ee, *ws)


def _call_upd(aggr_a, aggr_b, den_a, den_b, xn, x_in, *ws):
    grid = (N // BN,)
    nblk = pl.BlockSpec((BN, D), lambda i: (i, 0))
    ablk = pl.BlockSpec((2, BN, 128), lambda i: (0, i, 0))
    dblk = pl.BlockSpec((2, BN, 16), lambda i: (0, i, 0))
    wspecs = [_full(w.shape) for w in ws]
    return pl.pallas_call(
        _upd_body,
        grid=grid,
        in_specs=[ablk, ablk, dblk, dblk, nblk, nblk] + wspecs,
        out_specs=nblk,
        out_shape=jax.ShapeDtypeStruct((N, D), jnp.float32),
        compiler_params=_TC_PARAMS,
    )(aggr_a, aggr_b, den_a, den_b, xn, x_in, *ws)


def kernel(alg_embed, edge_index, positions, rotate_angles, rotate_mat, params):
    src = edge_index[0]
    dst = edge_index[1]
    src_h = (src[:EH], src[EH:])
    dst_h = (dst[:EH], dst[EH:])

    tbl = jnp.concatenate(
        [positions[:, HS - 1, :],
         jnp.cos(rotate_angles)[:, None], jnp.sin(rotate_angles)[:, None],
         rotate_mat.reshape(N, 4), jnp.zeros((N, 120), jnp.float32)], axis=1)

    enc = params["edge_enc"]
    rel = params["rel"]
    agg = params["aggr"]

    def r2(v):
        return v.reshape(1, -1)

    z = jnp.zeros((128, 128), jnp.float32)
    sa_m = z.at[jnp.array([0, 1, 0, 1, 2, 3, 3, 2]),
                jnp.array([0, 2, 4, 6, 8, 9, 10, 11])].set(1.0)
    sa2_m = z.at[jnp.array([0, 1, 0, 1]),
                 jnp.array([1, 3, 5, 7])].set(1.0)
    sb_m = z.at[jnp.array([4, 4, 6, 6, 5, 5, 7, 7, 2, 3, 2, 3]),
                jnp.array([0, 1, 2, 3, 4, 5, 6, 7, 8, 9, 10, 11])].set(1.0)
    scm = z.at[jnp.array([0, 2, 1, 3, 4, 6, 5, 7, 8, 9, 10, 11]),
               jnp.array([0, 0, 0, 0, 1, 1, 1, 1, 2, 2, 3, 3])].set(
        jnp.array([1., 1., -1., -1., 1., 1., -1., -1., 1., 1., 1., -1.]))
    w1e_ext = jnp.zeros((128, D), jnp.float32).at[0:2].set(enc["l1"]["w"])
    w1r_ext = jnp.zeros((128, D), jnp.float32).at[2:4].set(rel["l1"]["w"])
    we128 = scm @ w1e_ext
    wr128 = scm @ w1r_ext

    enc_ws = (
        sa_m, sa2_m, sb_m,
        we128, r2(enc["l1"]["b"]), r2(enc["ln"]["g"]), r2(enc["ln"]["b"]),
        enc["l2"]["w"].astype(jnp.bfloat16), r2(enc["l2"]["b"]),
        wr128, r2(rel["l1"]["b"]), r2(rel["ln"]["g"]), r2(rel["ln"]["b"]),
        rel["l2"]["w"].astype(jnp.bfloat16), r2(rel["l2"]["b"]),
        r2(agg["ln1"]["g"]), r2(agg["ln1"]["b"]),
        agg["lin"]["w"].astype(jnp.bfloat16), r2(agg["lin"]["b"]),
        r2(agg["ln2"]["g"]), r2(agg["ln2"]["b"]),
    )

    lp = params["layers"][-1]

    def cols(w, p):
        return w[:, p::2]

    def colb(b, p):
        return b[p::2].reshape(1, -1)

    def colsb(w, p):
        return w[:, p::2].astype(jnp.bfloat16)

    xn, q, kn, vn = _call_prep(
        alg_embed,
        cols(lp["q"]["w"], 0), cols(lp["q"]["w"], 1),
        colb(lp["q"]["b"], 0), colb(lp["q"]["b"], 1),
        cols(lp["kn"]["w"], 0), cols(lp["kn"]["w"], 1),
        colb(lp["kn"]["b"], 0), colb(lp["kn"]["b"], 1),
        cols(lp["vn"]["w"], 0), cols(lp["vn"]["w"], 1),
        colb(lp["vn"]["b"], 0), colb(lp["vn"]["b"], 1),
        r2(lp["norm1"]["g"]), r2(lp["norm1"]["b"]),
    )

    m16 = jnp.kron(jnp.eye(H, dtype=jnp.float32), jnp.ones((16, 16), jnp.float32))
    c16 = jnp.concatenate(
        [jnp.kron(jnp.eye(H, dtype=jnp.float32),
                  jnp.full((16, 1), 1.0 / 16, jnp.float32)),
         jnp.zeros((128, 120), jnp.float32)], axis=1)
    st128 = jnp.concatenate(
        [jnp.kron(jnp.eye(H, dtype=jnp.float32), jnp.ones((1, 16), jnp.float32)),
         jnp.zeros((8, 128), jnp.float32)], axis=0)

    att_ws = (
        colsb(lp["ke"]["w"], 0), colsb(lp["ke"]["w"], 1),
        colb(lp["ke"]["b"], 0), colb(lp["ke"]["b"], 1),
        colsb(lp["ve"]["w"], 0), colsb(lp["ve"]["w"], 1),
        colb(lp["ve"]["b"], 0), colb(lp["ve"]["b"], 1),
        m16, c16,
    )

    zn = jnp.zeros((N, 128), jnp.float32)
    zd = jnp.zeros((N, 16), jnp.float32)

    aggr_h = []
    den_h = []
    for hf in range(2):
        fs, fd = _gather_feats_h(tbl, src_h[hf], dst_h[hf])
        ee = _call_enc(fs, fd, *enc_ws)
        qd, ks, vs = _gather_qkv_h(q, kn, vn, src_h[hf], dst_h[hf])
        msg, ex = _call_att(qd, ks, vs, ee, *att_ws)
        aggr_h.append(_scatter_h(dst_h[hf], msg, zn))
        den_h.append(_scatter_den_h(dst_h[hf], ex, zd))

    def rows_(w, p):
        return w[p::2, :]

    bg = lp["ih"]["b"] + lp["hh"]["b"]
    out = _call_upd(
        aggr_h[0], aggr_h[1], den_h[0], den_h[1], xn, alg_embed, st128,
        rows_(cols(lp["ih"]["w"], 0), 0), rows_(cols(lp["ih"]["w"], 0), 1),
        rows_(cols(lp["ih"]["w"], 1), 0), rows_(cols(lp["ih"]["w"], 1), 1),
        cols(lp["hh"]["w"], 0), cols(lp["hh"]["w"], 1),
        colb(bg, 0), colb(bg, 1),
        cols(lp["self"]["w"], 0), cols(lp["self"]["w"], 1),
        colb(lp["self"]["b"], 0), colb(lp["self"]["b"], 1),
        rows_(lp["out"]["w"], 0), rows_(lp["out"]["w"], 1), r2(lp["out"]["b"]),
        r2(lp["norm2"]["g"]), r2(lp["norm2"]["b"]),
        lp["mlp1"]["w"], r2(lp["mlp1"]["b"]), lp["mlp2"]["w"], r2(lp["mlp2"]["b"]),
        r2(params["norm"]["g"]), r2(params["norm"]["b"]),
    )
    return out


# speedup vs baseline: 1.0385x; 1.0385x over previous
import jax
import jax.numpy as jnp
from jax import lax
from jax.experimental import pallas as pl
from jax.experimental.pallas import tpu as pltpu
from jax.experimental.pallas import tpu_sc as plsc

HS = 20
D = 256
H = 8
DH = D // H
N = 10000
E = 160000
EH = E // 2

NC = 2
NS = 16
NW = NC * NS

CH = 128
BE = 1280
BN = 2000
NPT = N // NS
SLAB = 632
SLAB_LAST = N - 15 * SLAB

_mesh = plsc.VectorSubcoreMesh(core_axis_name="c", subcore_axis_name="s")
_SC_UNTILED = pltpu.CompilerParams(use_tc_tiling_on_sc=False)


def _make_gather_feats(ecnt):
    nchunk = ecnt // CH

    def body_fn(tbl, src, dst, fs_out, fd_out,
                idx_s, idx_d, rows, gs0, gs1, ws0, ws1):
        wid = lax.axis_index("s") * NC + lax.axis_index("c")
        iters = (nchunk + NW - 1) // NW
        outs = (fs_out, fd_out)
        gsems = (gs0, gs1)
        wsems = (ws0, ws1)
        idxs = (idx_s, idx_d)

        def body(j, _):
            cid = j * NW + wid

            @pl.when(cid < nchunk)
            def _():
                base = cid * CH

                @pl.when(j > 0)
                def _():
                    for i in range(2):
                        pltpu.make_async_copy(
                            rows.at[i], outs[i].at[pl.ds(base, CH)], wsems[i]
                        ).wait()

                pltpu.sync_copy(src.at[pl.ds(base, CH)], idx_s)
                pltpu.sync_copy(dst.at[pl.ds(base, CH)], idx_d)
                for i in range(2):
                    pltpu.async_copy(tbl.at[idxs[i]], rows.at[i], gsems[i])
                for i in range(2):
                    pltpu.make_async_copy(tbl.at[idxs[i]], rows.at[i],
                                          gsems[i]).wait()
                    pltpu.async_copy(rows.at[i], outs[i].at[pl.ds(base, CH)],
                                     wsems[i])

            return 0

        lax.fori_loop(0, iters, body, 0)
        for i in range(2):
            pltpu.make_async_copy(rows.at[i], outs[i].at[pl.ds(0, CH)],
                                  wsems[i]).wait()

    return pl.kernel(
        body_fn,
        out_type=[
            jax.ShapeDtypeStruct((ecnt, 128), jnp.float32),
            jax.ShapeDtypeStruct((ecnt, 128), jnp.float32),
        ],
        mesh=_mesh,
        scratch_types=[
            pltpu.VMEM((CH,), jnp.int32),
            pltpu.VMEM((CH,), jnp.int32),
            pltpu.VMEM((2, CH, 128), jnp.float32),
            pltpu.SemaphoreType.DMA,
            pltpu.SemaphoreType.DMA,
            pltpu.SemaphoreType.DMA,
            pltpu.SemaphoreType.DMA,
        ],
    )


def _make_gather_qkv(ecnt):
    nchunk = ecnt // CH

    def body_fn(q, kn, vn, src, dst, qd_out, ks_out, vs_out,
                idx_s, idx_d, rows, gs0, gs1, gs2, ws0, ws1, ws2):
        wid = lax.axis_index("s") * NC + lax.axis_index("c")
        iters = (nchunk + NW - 1) // NW
        outs = (qd_out, ks_out, vs_out)
        gsems = (gs0, gs1, gs2)
        wsems = (ws0, ws1, ws2)
        tbls = (q, kn, vn)

        def body(j, _):
            cid = j * NW + wid

            @pl.when(cid < nchunk)
            def _():
                base = cid * CH

                @pl.when(j > 0)
                def _():
                    for i in range(3):
                        pltpu.make_async_copy(
                            rows.at[i], outs[i].at[pl.ds(base, CH)], wsems[i]
                        ).wait()

                pltpu.sync_copy(src.at[pl.ds(base, CH)], idx_s)
                pltpu.sync_copy(dst.at[pl.ds(base, CH)], idx_d)
                idxs = (idx_d, idx_s, idx_s)
                for i in range(3):
                    pltpu.async_copy(tbls[i].at[idxs[i]], rows.at[i], gsems[i])
                for i in range(3):
                    pltpu.make_async_copy(tbls[i].at[idxs[i]], rows.at[i],
                                          gsems[i]).wait()
                    pltpu.async_copy(rows.at[i], outs[i].at[pl.ds(base, CH)],
                                     wsems[i])

            return 0

        lax.fori_loop(0, iters, body, 0)
        for i in range(3):
            pltpu.make_async_copy(rows.at[i], outs[i].at[pl.ds(0, CH)],
                                  wsems[i]).wait()

    return pl.kernel(
        body_fn,
        out_type=[
            jax.ShapeDtypeStruct((ecnt, 128), jnp.int32),
            jax.ShapeDtypeStruct((ecnt, 128), jnp.int32),
            jax.ShapeDtypeStruct((ecnt, 128), jnp.int32),
        ],
        mesh=_mesh,
        scratch_types=[
            pltpu.VMEM((CH,), jnp.int32),
            pltpu.VMEM((CH,), jnp.int32),
            pltpu.VMEM((3, CH, 128), jnp.int32),
            pltpu.SemaphoreType.DMA,
            pltpu.SemaphoreType.DMA,
            pltpu.SemaphoreType.DMA,
            pltpu.SemaphoreType.DMA,
            pltpu.SemaphoreType.DMA,
            pltpu.SemaphoreType.DMA,
        ],
    )


def _make_scatter(ecnt):
    nchunk = ecnt // CH

    def body_fn(dst, msg, zn, aggr_out, shared, idx_b, mrows, ms0, ms1, lsem):
        c = lax.axis_index("c")
        s = lax.axis_index("s")
        msems = (ms0, ms1)

        @pl.when(s < 15)
        def _():
            pltpu.sync_copy(zn.at[pl.ds(s * SLAB, SLAB)],
                            shared.at[pl.ds(s * SLAB, SLAB)])

        @pl.when(s == 15)
        def _():
            pltpu.sync_copy(zn.at[pl.ds(15 * SLAB, SLAB_LAST)],
                            shared.at[pl.ds(15 * SLAB, SLAB_LAST)])

        plsc.subcore_barrier()

        iters = (nchunk + NS - 1) // NS

        def body(jj, _):
            for p in range(2):
                j = jj * 2 + p
                cid = j * NS + s

                @pl.when(cid < nchunk)
                def _():
                    base = cid * CH

                    @pl.when(j > 1)
                    def _():
                        pltpu.make_async_copy(
                            mrows.at[p], shared.at[idx_b.at[p]],
                            msems[p]).wait()

                    pltpu.sync_copy(dst.at[pl.ds(base, CH)], idx_b.at[p])
                    pltpu.async_copy(msg.at[c, pl.ds(base, CH)], mrows.at[p],
                                     lsem)
                    pltpu.make_async_copy(msg.at[c, pl.ds(base, CH)],
                                          mrows.at[p], lsem).wait()
                    pltpu.async_copy(mrows.at[p], shared.at[idx_b.at[p]],
                                     msems[p], add=True)

            return 0

        lax.fori_loop(0, (iters + 1) // 2, body, 0)
        for p in range(2):
            pltpu.make_async_copy(mrows.at[p], shared.at[idx_b.at[p]],
                                  msems[p]).wait()

        plsc.subcore_barrier()

        @pl.when(s < 15)
        def _():
            pltpu.sync_copy(shared.at[pl.ds(s * SLAB, SLAB)],
                            aggr_out.at[c, pl.ds(s * SLAB, SLAB)])

        @pl.when(s == 15)
        def _():
            pltpu.sync_copy(shared.at[pl.ds(15 * SLAB, SLAB_LAST)],
                            aggr_out.at[c, pl.ds(15 * SLAB, SLAB_LAST)])

    return pl.kernel(
        body_fn,
        out_type=jax.ShapeDtypeStruct((NC, N, 128), jnp.float32),
        mesh=_mesh,
        scratch_types=[
            pltpu.VMEM_SHARED((N, 128), jnp.float32),
            pltpu.VMEM((2, CH), jnp.int32),
            pltpu.VMEM((2, CH, 128), jnp.float32),
            pltpu.SemaphoreType.DMA,
            pltpu.SemaphoreType.DMA,
            pltpu.SemaphoreType.DMA,
        ],
    )


def _make_scatter_den(ecnt):
    nchunk = ecnt // CH

    def body_fn(dst, ex, zd, den_out, shared_den, idx_b, erows,
                ds0, ds1, lsem):
        c = lax.axis_index("c")
        s = lax.axis_index("s")
        wid = s * NC + c
        dsems = (ds0, ds1)

        pltpu.sync_copy(zd.at[pl.ds(s * NPT, NPT)],
                        shared_den.at[pl.ds(s * NPT, NPT)])
        plsc.subcore_barrier()

        iters = (nchunk + NW - 1) // NW

        def body(jj, _):
            for p in range(2):
                j = jj * 2 + p
                cid = j * NW + wid

                @pl.when(cid < nchunk)
                def _():
                    base = cid * CH

                    @pl.when(j > 1)
                    def _():
                        pltpu.make_async_copy(
                            erows.at[p], shared_den.at[idx_b.at[p]],
                            dsems[p]).wait()

                    pltpu.sync_copy(dst.at[pl.ds(base, CH)], idx_b.at[p])
                    exs = ex.at[pl.ds(base, CH), pl.ds(0, 16)]
                    pltpu.async_copy(exs, erows.at[p], lsem)
                    pltpu.make_async_copy(exs, erows.at[p], lsem).wait()
                    pltpu.async_copy(erows.at[p], shared_den.at[idx_b.at[p]],
                                     dsems[p], add=True)

            return 0

        lax.fori_loop(0, (iters + 1) // 2, body, 0)
        for p in range(2):
            pltpu.make_async_copy(erows.at[p], shared_den.at[idx_b.at[p]],
                                  dsems[p]).wait()

        plsc.subcore_barrier()
        pltpu.sync_copy(shared_den.at[pl.ds(s * NPT, NPT)],
                        den_out.at[c, pl.ds(s * NPT, NPT)])

    return pl.kernel(
        body_fn,
        out_type=jax.ShapeDtypeStruct((NC, N, 16), jnp.float32),
        mesh=_mesh,
        scratch_types=[
            pltpu.VMEM_SHARED((N, 16), jnp.float32),
            pltpu.VMEM((2, CH), jnp.int32),
            pltpu.VMEM((2, CH, 16), jnp.float32),
            pltpu.SemaphoreType.DMA,
            pltpu.SemaphoreType.DMA,
            pltpu.SemaphoreType.DMA,
        ],
        compiler_params=_SC_UNTILED,
    )


_gather_feats_f = _make_gather_feats(E)
_gather_qkv_f = _make_gather_qkv(E)
_scatter_f = _make_scatter(E)
_scatter_den_f = _make_scatter_den(E)


def _ln_in(x, g, b):
    m = jnp.mean(x, axis=-1, keepdims=True)
    v = jnp.mean((x - m) ** 2, axis=-1, keepdims=True)
    return (x - m) * lax.rsqrt(v + 1e-5) * g + b


def _dot(a, b):
    return jnp.dot(a, b, preferred_element_type=jnp.float32)


def _pack_bf16(even, odd):
    ue = lax.bitcast_convert_type(even, jnp.uint32)
    ue = ue + jnp.uint32(0x7FFF) + ((ue >> jnp.uint32(16)) & jnp.uint32(1))
    uo = lax.bitcast_convert_type(odd, jnp.uint32)
    uo = uo + jnp.uint32(0x7FFF) + ((uo >> jnp.uint32(16)) & jnp.uint32(1))
    packed = (uo & jnp.uint32(0xFFFF0000)) | (ue >> jnp.uint32(16))
    return lax.bitcast_convert_type(packed, jnp.int32)


def _unpack_bf16(xi):
    u = lax.bitcast_convert_type(xi, jnp.uint32)
    even = lax.bitcast_convert_type(u << jnp.uint32(16), jnp.float32)
    odd = lax.bitcast_convert_type(u & jnp.uint32(0xFFFF0000), jnp.float32)
    return even, odd


def _enc_body(fs, fd, sa_m, sa2_m, sb_m, we128, b1e, ge, be_, w2e, b2e,
              wr128, b1r, gr, br, w2r, b2r,
              ga1, ba1, wa, ba, ga2, ba2, ee_out):
    fs_ = fs[...]
    fd_ = fd[...]
    lft = _dot(fs_, sa_m[...]) + _dot(fd_, sa2_m[...])
    rgt = _dot(fd_, sb_m[...])
    prods = lft * rgt
    e = _dot(prods, we128[...]) + b1e[...]
    e = jnp.maximum(_ln_in(e, ge[...], be_[...]), 0.0).astype(jnp.bfloat16)
    e = _dot(e, w2e[...]) + b2e[...]
    r = _dot(prods, wr128[...]) + b1r[...]
    r = jnp.maximum(_ln_in(r, gr[...], br[...]), 0.0).astype(jnp.bfloat16)
    r = _dot(r, w2r[...]) + b2r[...]
    ee = e + r
    ee = jnp.maximum(_ln_in(ee, ga1[...], ba1[...]), 0.0).astype(jnp.bfloat16)
    ee = _dot(ee, wa[...]) + ba[...]
    ee_out[...] = _ln_in(ee, ga2[...], ba2[...]).astype(jnp.bfloat16)


def _prep_body(x, wq_e, wq_o, bq_e, bq_o, wk_e, wk_o, bk_e, bk_o,
               wv_e, wv_o, bv_e, bv_o, g1, b1,
               xn_out, q_out, k_out, v_out):
    xn = _ln_in(x[...], g1[...], b1[...])
    xn_out[...] = xn
    q_out[...] = _pack_bf16(_dot(xn, wq_e[...]) + bq_e[...],
                            _dot(xn, wq_o[...]) + bq_o[...])
    k_out[...] = _pack_bf16(_dot(xn, wk_e[...]) + bk_e[...],
                            _dot(xn, wk_o[...]) + bk_o[...])
    v_out[...] = _pack_bf16(_dot(xn, wv_e[...]) + bv_e[...],
                            _dot(xn, wv_o[...]) + bv_o[...])


def _att_body(qd, ks, vs, ee, wke_e, wke_o, bke_e, bke_o,
              wve_e, wve_o, bve_e, bve_o, m16, c16,
              msg_out, ex_out):
    ee_ = ee[...]
    qde, qdo = _unpack_bf16(qd[...])
    kse, kso = _unpack_bf16(ks[...])
    vse, vso = _unpack_bf16(vs[...])
    ke_e = _dot(ee_, wke_e[...]) + bke_e[...]
    ke_o = _dot(ee_, wke_o[...]) + bke_o[...]
    prod = qde * (kse + ke_e) + qdo * (kso + ke_o)
    s = _dot(prod, m16[...])
    ex = jnp.exp(s * (1.0 / (DH ** 0.5)))
    ve_e = _dot(ee_, wve_e[...]) + bve_e[...]
    ve_o = _dot(ee_, wve_o[...]) + bve_o[...]
    msg_out[0] = (vse + ve_e) * ex
    msg_out[1] = (vso + ve_o) * ex
    ex_out[...] = _dot(ex, c16[...])


def _upd_body(aggr, den, xn, x_in, st128,
              wih_ee, wih_oe, wih_eo, wih_oo, whh_e, whh_o, bg_e, bg_o,
              wsf_e, wsf_o, bsf_e, bsf_o, wo_e, wo_o, bo,
              g2, b2, wm1, bm1, wm2, bm2, gf, bf, out):
    den_e = _dot(den[0] + den[1], st128[...])
    rcp = 1.0 / (den_e + 1e-16)
    a_e = aggr[0] * rcp
    a_o = aggr[1] * rcp
    xn_ = xn[...]
    gate_e = jax.nn.sigmoid(_dot(a_e, wih_ee[...]) + _dot(a_o, wih_oe[...])
                            + _dot(xn_, whh_e[...]) + bg_e[...])
    gate_o = jax.nn.sigmoid(_dot(a_e, wih_eo[...]) + _dot(a_o, wih_oo[...])
                            + _dot(xn_, whh_o[...]) + bg_o[...])
    sf_e = _dot(xn_, wsf_e[...]) + bsf_e[...]
    sf_o = _dot(xn_, wsf_o[...]) + bsf_o[...]
    upd_e = a_e + gate_e * (sf_e - a_e)
    upd_o = a_o + gate_o * (sf_o - a_o)
    x = x_in[...] + _dot(upd_e, wo_e[...]) + _dot(upd_o, wo_o[...]) + bo[...]
    xn2 = _ln_in(x, g2[...], b2[...])
    h = jnp.maximum(_dot(xn2, wm1[...]) + bm1[...], 0.0)
    h = _dot(h, wm2[...]) + bm2[...]
    out[...] = _ln_in(x + h, gf[...], bf[...])


def _full(shape):
    nd = len(shape)
    return pl.BlockSpec(shape, lambda i: (0,) * nd)


_TC_PARAMS = pltpu.CompilerParams(
    dimension_semantics=("arbitrary",),
)


def _call_enc(fs, fd, *ws):
    ecnt = fs.shape[0]
    grid = (ecnt // BE,)
    especs = [pl.BlockSpec((BE, 128), lambda i: (i, 0))] * 2
    wspecs = [_full(w.shape) for w in ws]
    return pl.pallas_call(
        _enc_body,
        grid=grid,
        in_specs=especs + wspecs,
        out_specs=pl.BlockSpec((BE, D), lambda i: (i, 0)),
        out_shape=jax.ShapeDtypeStruct((ecnt, D), jnp.bfloat16),
        compiler_params=_TC_PARAMS,
    )(fs, fd, *ws)


def _call_prep(x, *ws):
    grid = (N // BN,)
    wspecs = [_full(w.shape) for w in ws]
    blk = pl.BlockSpec((BN, D), lambda i: (i, 0))
    hblk = pl.BlockSpec((BN, 128), lambda i: (i, 0))
    return pl.pallas_call(
        _prep_body,
        grid=grid,
        in_specs=[blk] + wspecs,
        out_specs=[blk] + [hblk] * 3,
        out_shape=[jax.ShapeDtypeStruct((N, D), jnp.float32)]
        + [jax.ShapeDtypeStruct((N, 128), jnp.int32)] * 3,
        compiler_params=_TC_PARAMS,
    )(x, *ws)


def _call_att(qd, ks, vs, ee, *ws):
    ecnt = qd.shape[0]
    grid = (ecnt // BE,)
    hblk = pl.BlockSpec((BE, 128), lambda i: (i, 0))
    eblk = pl.BlockSpec((BE, D), lambda i: (i, 0))
    wspecs = [_full(w.shape) for w in ws]
    return pl.pallas_call(
        _att_body,
        grid=grid,
        in_specs=[hblk] * 3 + [eblk] + wspecs,
        out_specs=[
            pl.BlockSpec((2, BE, 128), lambda i: (0, i, 0)),
            pl.BlockSpec((BE, 128), lambda i: (i, 0)),
        ],
        out_shape=[
            jax.ShapeDtypeStruct((2, ecnt, 128), jnp.float32),
            jax.ShapeDtypeStruct((ecnt, 128), jnp.float32),
        ],
        compiler_params=_TC_PARAMS,
    )(qd, ks, vs, ee, *ws)


def _call_upd(aggr, den, xn, x_in, *ws):
    grid = (N // BN,)
    nblk = pl.BlockSpec((BN, D), lambda i: (i, 0))
    ablk = pl.BlockSpec((2, BN, 128), lambda i: (0, i, 0))
    dblk = pl.BlockSpec((2, BN, 16), lambda i: (0, i, 0))
    wspecs = [_full(w.shape) for w in ws]
    return pl.pallas_call(
        _upd_body,
        grid=grid,
        in_specs=[ablk, dblk, nblk, nblk] + wspecs,
        out_specs=nblk,
        out_shape=jax.ShapeDtypeStruct((N, D), jnp.float32),
        compiler_params=_TC_PARAMS,
    )(aggr, den, xn, x_in, *ws)


def kernel(alg_embed, edge_index, positions, rotate_angles, rotate_mat, params):
    src = edge_index[0]
    dst = edge_index[1]

    tbl = jnp.concatenate(
        [positions[:, HS - 1, :],
         jnp.cos(rotate_angles)[:, None], jnp.sin(rotate_angles)[:, None],
         rotate_mat.reshape(N, 4), jnp.zeros((N, 120), jnp.float32)], axis=1)

    enc = params["edge_enc"]
    rel = params["rel"]
    agg = params["aggr"]

    def r2(v):
        return v.reshape(1, -1)

    z = jnp.zeros((128, 128), jnp.float32)
    sa_m = z.at[jnp.array([0, 1, 0, 1, 2, 3, 3, 2]),
                jnp.array([0, 2, 4, 6, 8, 9, 10, 11])].set(1.0)
    sa2_m = z.at[jnp.array([0, 1, 0, 1]),
                 jnp.array([1, 3, 5, 7])].set(1.0)
    sb_m = z.at[jnp.array([4, 4, 6, 6, 5, 5, 7, 7, 2, 3, 2, 3]),
                jnp.array([0, 1, 2, 3, 4, 5, 6, 7, 8, 9, 10, 11])].set(1.0)
    scm = z.at[jnp.array([0, 2, 1, 3, 4, 6, 5, 7, 8, 9, 10, 11]),
               jnp.array([0, 0, 0, 0, 1, 1, 1, 1, 2, 2, 3, 3])].set(
        jnp.array([1., 1., -1., -1., 1., 1., -1., -1., 1., 1., 1., -1.]))
    w1e_ext = jnp.zeros((128, D), jnp.float32).at[0:2].set(enc["l1"]["w"])
    w1r_ext = jnp.zeros((128, D), jnp.float32).at[2:4].set(rel["l1"]["w"])
    we128 = scm @ w1e_ext
    wr128 = scm @ w1r_ext

    enc_ws = (
        sa_m, sa2_m, sb_m,
        we128, r2(enc["l1"]["b"]), r2(enc["ln"]["g"]), r2(enc["ln"]["b"]),
        enc["l2"]["w"].astype(jnp.bfloat16), r2(enc["l2"]["b"]),
        wr128, r2(rel["l1"]["b"]), r2(rel["ln"]["g"]), r2(rel["ln"]["b"]),
        rel["l2"]["w"].astype(jnp.bfloat16), r2(rel["l2"]["b"]),
        r2(agg["ln1"]["g"]), r2(agg["ln1"]["b"]),
        agg["lin"]["w"].astype(jnp.bfloat16), r2(agg["lin"]["b"]),
        r2(agg["ln2"]["g"]), r2(agg["ln2"]["b"]),
    )

    lp = params["layers"][-1]

    def cols(w, p):
        return w[:, p::2]

    def colb(b, p):
        return b[p::2].reshape(1, -1)

    def colsb(w, p):
        return w[:, p::2].astype(jnp.bfloat16)

    xn, q, kn, vn = _call_prep(
        alg_embed,
        cols(lp["q"]["w"], 0), cols(lp["q"]["w"], 1),
        colb(lp["q"]["b"], 0), colb(lp["q"]["b"], 1),
        cols(lp["kn"]["w"], 0), cols(lp["kn"]["w"], 1),
        colb(lp["kn"]["b"], 0), colb(lp["kn"]["b"], 1),
        cols(lp["vn"]["w"], 0), cols(lp["vn"]["w"], 1),
        colb(lp["vn"]["b"], 0), colb(lp["vn"]["b"], 1),
        r2(lp["norm1"]["g"]), r2(lp["norm1"]["b"]),
    )

    m16 = jnp.kron(jnp.eye(H, dtype=jnp.float32), jnp.ones((16, 16), jnp.float32))
    c16 = jnp.concatenate(
        [jnp.kron(jnp.eye(H, dtype=jnp.float32),
                  jnp.full((16, 1), 1.0 / 16, jnp.float32)),
         jnp.zeros((128, 120), jnp.float32)], axis=1)
    st128 = jnp.concatenate(
        [jnp.kron(jnp.eye(H, dtype=jnp.float32), jnp.ones((1, 16), jnp.float32)),
         jnp.zeros((8, 128), jnp.float32)], axis=0)

    att_ws = (
        colsb(lp["ke"]["w"], 0), colsb(lp["ke"]["w"], 1),
        colb(lp["ke"]["b"], 0), colb(lp["ke"]["b"], 1),
        colsb(lp["ve"]["w"], 0), colsb(lp["ve"]["w"], 1),
        colb(lp["ve"]["b"], 0), colb(lp["ve"]["b"], 1),
        m16, c16,
    )

    zn = jnp.zeros((N, 128), jnp.float32)
    zd = jnp.zeros((N, 16), jnp.float32)

    fs, fd = _gather_feats_f(tbl, src, dst)
    ee = _call_enc(fs, fd, *enc_ws)
    qd, ks, vs = _gather_qkv_f(q, kn, vn, src, dst)
    msg, ex = _call_att(qd, ks, vs, ee, *att_ws)
    aggr = _scatter_f(dst, msg, zn)
    den = _scatter_den_f(dst, ex, zd)

    def rows_(w, p):
        return w[p::2, :]

    bg = lp["ih"]["b"] + lp["hh"]["b"]
    out = _call_upd(
        aggr, den, xn, alg_embed, st128,
        rows_(cols(lp["ih"]["w"], 0), 0), rows_(cols(lp["ih"]["w"], 0), 1),
        rows_(cols(lp["ih"]["w"], 1), 0), rows_(cols(lp["ih"]["w"], 1), 1),
        cols(lp["hh"]["w"], 0), cols(lp["hh"]["w"], 1),
        colb(bg, 0), colb(bg, 1),
        cols(lp["self"]["w"], 0), cols(lp["self"]["w"], 1),
        colb(lp["self"]["b"], 0), colb(lp["self"]["b"], 1),
        rows_(lp["out"]["w"], 0), rows_(lp["out"]["w"], 1), r2(lp["out"]["b"]),
        r2(lp["norm2"]["g"]), r2(lp["norm2"]["b"]),
        lp["mlp1"]["w"], r2(lp["mlp1"]["b"]), lp["mlp2"]["w"], r2(lp["mlp2"]["b"]),
        r2(params["norm"]["g"]), r2(params["norm"]["b"]),
    )
    return out


# speedup vs baseline: 1.1036x; 1.0627x over previous
import jax
import jax.numpy as jnp
from jax import lax
from jax.experimental import pallas as pl
from jax.experimental.pallas import tpu as pltpu
from jax.experimental.pallas import tpu_sc as plsc

HS = 20
D = 256
H = 8
DH = D // H
N = 10000
E = 160000
EH = E // 2

NC = 2
NS = 16
NW = NC * NS

CH = 128
BE = 2560
BN = 2000
NPT = N // NS
SLAB = 632
SLAB_LAST = N - 15 * SLAB

_mesh = plsc.VectorSubcoreMesh(core_axis_name="c", subcore_axis_name="s")
_SC_UNTILED = pltpu.CompilerParams(use_tc_tiling_on_sc=False)


def _make_gather_feats(ecnt):
    nchunk = ecnt // CH

    def body_fn(tbl, src, dst, fs_out, fd_out,
                idx_s, idx_d, rows, gs0, gs1, ws0, ws1):
        wid = lax.axis_index("s") * NC + lax.axis_index("c")
        iters = (nchunk + NW - 1) // NW
        outs = (fs_out, fd_out)
        gsems = (gs0, gs1)
        wsems = (ws0, ws1)
        idxs = (idx_s, idx_d)

        def body(j, _):
            cid = j * NW + wid

            @pl.when(cid < nchunk)
            def _():
                base = cid * CH

                @pl.when(j > 0)
                def _():
                    for i in range(2):
                        pltpu.make_async_copy(
                            rows.at[i], outs[i].at[pl.ds(base, CH)], wsems[i]
                        ).wait()

                pltpu.sync_copy(src.at[pl.ds(base, CH)], idx_s)
                pltpu.sync_copy(dst.at[pl.ds(base, CH)], idx_d)
                for i in range(2):
                    pltpu.async_copy(tbl.at[idxs[i]], rows.at[i], gsems[i])
                for i in range(2):
                    pltpu.make_async_copy(tbl.at[idxs[i]], rows.at[i],
                                          gsems[i]).wait()
                    pltpu.async_copy(rows.at[i], outs[i].at[pl.ds(base, CH)],
                                     wsems[i])

            return 0

        lax.fori_loop(0, iters, body, 0)
        for i in range(2):
            pltpu.make_async_copy(rows.at[i], outs[i].at[pl.ds(0, CH)],
                                  wsems[i]).wait()

    return pl.kernel(
        body_fn,
        out_type=[
            jax.ShapeDtypeStruct((ecnt, 128), jnp.float32),
            jax.ShapeDtypeStruct((ecnt, 128), jnp.float32),
        ],
        mesh=_mesh,
        scratch_types=[
            pltpu.VMEM((CH,), jnp.int32),
            pltpu.VMEM((CH,), jnp.int32),
            pltpu.VMEM((2, CH, 128), jnp.float32),
            pltpu.SemaphoreType.DMA,
            pltpu.SemaphoreType.DMA,
            pltpu.SemaphoreType.DMA,
            pltpu.SemaphoreType.DMA,
        ],
    )


def _make_gather_qkv(ecnt):
    nchunk = ecnt // CH

    def body_fn(q, kn, vn, src, dst, qd_out, ks_out, vs_out,
                idx_s, idx_d, rows, gs0, gs1, gs2, ws0, ws1, ws2):
        wid = lax.axis_index("s") * NC + lax.axis_index("c")
        iters = (nchunk + NW - 1) // NW
        outs = (qd_out, ks_out, vs_out)
        gsems = (gs0, gs1, gs2)
        wsems = (ws0, ws1, ws2)
        tbls = (q, kn, vn)

        def body(j, _):
            cid = j * NW + wid

            @pl.when(cid < nchunk)
            def _():
                base = cid * CH

                @pl.when(j > 0)
                def _():
                    for i in range(3):
                        pltpu.make_async_copy(
                            rows.at[i], outs[i].at[pl.ds(base, CH)], wsems[i]
                        ).wait()

                pltpu.sync_copy(src.at[pl.ds(base, CH)], idx_s)
                pltpu.sync_copy(dst.at[pl.ds(base, CH)], idx_d)
                idxs = (idx_d, idx_s, idx_s)
                for i in range(3):
                    pltpu.async_copy(tbls[i].at[idxs[i]], rows.at[i], gsems[i])
                for i in range(3):
                    pltpu.make_async_copy(tbls[i].at[idxs[i]], rows.at[i],
                                          gsems[i]).wait()
                    pltpu.async_copy(rows.at[i], outs[i].at[pl.ds(base, CH)],
                                     wsems[i])

            return 0

        lax.fori_loop(0, iters, body, 0)
        for i in range(3):
            pltpu.make_async_copy(rows.at[i], outs[i].at[pl.ds(0, CH)],
                                  wsems[i]).wait()

    return pl.kernel(
        body_fn,
        out_type=[
            jax.ShapeDtypeStruct((ecnt, 128), jnp.int32),
            jax.ShapeDtypeStruct((ecnt, 128), jnp.int32),
            jax.ShapeDtypeStruct((ecnt, 128), jnp.int32),
        ],
        mesh=_mesh,
        scratch_types=[
            pltpu.VMEM((CH,), jnp.int32),
            pltpu.VMEM((CH,), jnp.int32),
            pltpu.VMEM((3, CH, 128), jnp.int32),
            pltpu.SemaphoreType.DMA,
            pltpu.SemaphoreType.DMA,
            pltpu.SemaphoreType.DMA,
            pltpu.SemaphoreType.DMA,
            pltpu.SemaphoreType.DMA,
            pltpu.SemaphoreType.DMA,
        ],
    )


def _make_scatter(ecnt):
    nchunk = ecnt // CH

    def body_fn(dst, msg, zn, aggr_out, shared, idx_b, mrows, ms0, ms1, lsem):
        c = lax.axis_index("c")
        s = lax.axis_index("s")
        msems = (ms0, ms1)

        @pl.when(s < 15)
        def _():
            pltpu.sync_copy(zn.at[pl.ds(s * SLAB, SLAB)],
                            shared.at[pl.ds(s * SLAB, SLAB)])

        @pl.when(s == 15)
        def _():
            pltpu.sync_copy(zn.at[pl.ds(15 * SLAB, SLAB_LAST)],
                            shared.at[pl.ds(15 * SLAB, SLAB_LAST)])

        plsc.subcore_barrier()

        iters = (nchunk + NS - 1) // NS

        def body(jj, _):
            for p in range(2):
                j = jj * 2 + p
                cid = j * NS + s

                @pl.when(cid < nchunk)
                def _():
                    base = cid * CH

                    @pl.when(j > 1)
                    def _():
                        pltpu.make_async_copy(
                            mrows.at[p], shared.at[idx_b.at[p]],
                            msems[p]).wait()

                    pltpu.sync_copy(dst.at[pl.ds(base, CH)], idx_b.at[p])
                    pltpu.async_copy(msg.at[c, pl.ds(base, CH)], mrows.at[p],
                                     lsem)
                    pltpu.make_async_copy(msg.at[c, pl.ds(base, CH)],
                                          mrows.at[p], lsem).wait()
                    pltpu.async_copy(mrows.at[p], shared.at[idx_b.at[p]],
                                     msems[p], add=True)

            return 0

        lax.fori_loop(0, (iters + 1) // 2, body, 0)
        for p in range(2):
            pltpu.make_async_copy(mrows.at[p], shared.at[idx_b.at[p]],
                                  msems[p]).wait()

        plsc.subcore_barrier()

        @pl.when(s < 15)
        def _():
            pltpu.sync_copy(shared.at[pl.ds(s * SLAB, SLAB)],
                            aggr_out.at[c, pl.ds(s * SLAB, SLAB)])

        @pl.when(s == 15)
        def _():
            pltpu.sync_copy(shared.at[pl.ds(15 * SLAB, SLAB_LAST)],
                            aggr_out.at[c, pl.ds(15 * SLAB, SLAB_LAST)])

    return pl.kernel(
        body_fn,
        out_type=jax.ShapeDtypeStruct((NC, N, 128), jnp.float32),
        mesh=_mesh,
        scratch_types=[
            pltpu.VMEM_SHARED((N, 128), jnp.float32),
            pltpu.VMEM((2, CH), jnp.int32),
            pltpu.VMEM((2, CH, 128), jnp.float32),
            pltpu.SemaphoreType.DMA,
            pltpu.SemaphoreType.DMA,
            pltpu.SemaphoreType.DMA,
        ],
    )


def _make_scatter_den(ecnt):
    nchunk = ecnt // CH

    def body_fn(dst, ex, zd, den_out, shared_den, idx_b, erows,
                ds0, ds1, lsem):
        c = lax.axis_index("c")
        s = lax.axis_index("s")
        wid = s * NC + c
        dsems = (ds0, ds1)

        pltpu.sync_copy(zd.at[pl.ds(s * NPT, NPT)],
                        shared_den.at[pl.ds(s * NPT, NPT)])
        plsc.subcore_barrier()

        iters = (nchunk + NW - 1) // NW

        def body(jj, _):
            for p in range(2):
                j = jj * 2 + p
                cid = j * NW + wid

                @pl.when(cid < nchunk)
                def _():
                    base = cid * CH

                    @pl.when(j > 1)
                    def _():
                        pltpu.make_async_copy(
                            erows.at[p], shared_den.at[idx_b.at[p]],
                            dsems[p]).wait()

                    pltpu.sync_copy(dst.at[pl.ds(base, CH)], idx_b.at[p])
                    exs = ex.at[pl.ds(base, CH), pl.ds(0, 16)]
                    pltpu.async_copy(exs, erows.at[p], lsem)
                    pltpu.make_async_copy(exs, erows.at[p], lsem).wait()
                    pltpu.async_copy(erows.at[p], shared_den.at[idx_b.at[p]],
                                     dsems[p], add=True)

            return 0

        lax.fori_loop(0, (iters + 1) // 2, body, 0)
        for p in range(2):
            pltpu.make_async_copy(erows.at[p], shared_den.at[idx_b.at[p]],
                                  dsems[p]).wait()

        plsc.subcore_barrier()
        pltpu.sync_copy(shared_den.at[pl.ds(s * NPT, NPT)],
                        den_out.at[c, pl.ds(s * NPT, NPT)])

    return pl.kernel(
        body_fn,
        out_type=jax.ShapeDtypeStruct((NC, N, 16), jnp.float32),
        mesh=_mesh,
        scratch_types=[
            pltpu.VMEM_SHARED((N, 16), jnp.float32),
            pltpu.VMEM((2, CH), jnp.int32),
            pltpu.VMEM((2, CH, 16), jnp.float32),
            pltpu.SemaphoreType.DMA,
            pltpu.SemaphoreType.DMA,
            pltpu.SemaphoreType.DMA,
        ],
        compiler_params=_SC_UNTILED,
    )


_gather_feats_f = _make_gather_feats(E)
_gather_qkv_f = _make_gather_qkv(E)
_scatter_f = _make_scatter(E)
_scatter_den_f = _make_scatter_den(E)


def _ln_in(x, g, b):
    m = jnp.mean(x, axis=-1, keepdims=True)
    v = jnp.mean((x - m) ** 2, axis=-1, keepdims=True)
    return (x - m) * lax.rsqrt(v + 1e-5) * g + b


def _dot(a, b):
    return jnp.dot(a, b, preferred_element_type=jnp.float32)


def _pack_bf16(even, odd):
    ue = lax.bitcast_convert_type(even, jnp.uint32)
    ue = ue + jnp.uint32(0x7FFF) + ((ue >> jnp.uint32(16)) & jnp.uint32(1))
    uo = lax.bitcast_convert_type(odd, jnp.uint32)
    uo = uo + jnp.uint32(0x7FFF) + ((uo >> jnp.uint32(16)) & jnp.uint32(1))
    packed = (uo & jnp.uint32(0xFFFF0000)) | (ue >> jnp.uint32(16))
    return lax.bitcast_convert_type(packed, jnp.int32)


def _unpack_bf16(xi):
    u = lax.bitcast_convert_type(xi, jnp.uint32)
    even = lax.bitcast_convert_type(u << jnp.uint32(16), jnp.float32)
    odd = lax.bitcast_convert_type(u & jnp.uint32(0xFFFF0000), jnp.float32)
    return even, odd


def _enc_body(fs, fd, sa_m, sa2_m, sb_m, we128, b1e, ge, be_, w2e, b2e,
              wr128, b1r, gr, br, w2r, b2r,
              ga1, ba1, wa, ba, ga2, ba2, ee_out):
    fs_ = fs[...]
    fd_ = fd[...]
    lft = _dot(fs_, sa_m[...]) + _dot(fd_, sa2_m[...])
    rgt = _dot(fd_, sb_m[...])
    prods = lft * rgt
    e = _dot(prods, we128[...]) + b1e[...]
    e = jnp.maximum(_ln_in(e, ge[...], be_[...]), 0.0).astype(jnp.bfloat16)
    e = _dot(e, w2e[...]) + b2e[...]
    r = _dot(prods, wr128[...]) + b1r[...]
    r = jnp.maximum(_ln_in(r, gr[...], br[...]), 0.0).astype(jnp.bfloat16)
    r = _dot(r, w2r[...]) + b2r[...]
    ee = e + r
    ee = jnp.maximum(_ln_in(ee, ga1[...], ba1[...]), 0.0).astype(jnp.bfloat16)
    ee = _dot(ee, wa[...]) + ba[...]
    ee_out[...] = _ln_in(ee, ga2[...], ba2[...]).astype(jnp.bfloat16)


def _prep_body(x, wq_e, wq_o, bq_e, bq_o, wk_e, wk_o, bk_e, bk_o,
               wv_e, wv_o, bv_e, bv_o, g1, b1,
               xn_out, q_out, k_out, v_out):
    xn = _ln_in(x[...], g1[...], b1[...])
    xn_out[...] = xn
    q_out[...] = _pack_bf16(_dot(xn, wq_e[...]) + bq_e[...],
                            _dot(xn, wq_o[...]) + bq_o[...])
    k_out[...] = _pack_bf16(_dot(xn, wk_e[...]) + bk_e[...],
                            _dot(xn, wk_o[...]) + bk_o[...])
    v_out[...] = _pack_bf16(_dot(xn, wv_e[...]) + bv_e[...],
                            _dot(xn, wv_o[...]) + bv_o[...])


def _att_body(qd, ks, vs, ee, wke_e, wke_o, bke_e, bke_o,
              wve_e, wve_o, bve_e, bve_o, m16, c16,
              msg_out, ex_out):
    ee_ = ee[...]
    qde, qdo = _unpack_bf16(qd[...])
    kse, kso = _unpack_bf16(ks[...])
    vse, vso = _unpack_bf16(vs[...])
    ke_e = _dot(ee_, wke_e[...]) + bke_e[...]
    ke_o = _dot(ee_, wke_o[...]) + bke_o[...]
    prod = qde * (kse + ke_e) + qdo * (kso + ke_o)
    s = _dot(prod, m16[...])
    ex = jnp.exp(s * (1.0 / (DH ** 0.5)))
    ve_e = _dot(ee_, wve_e[...]) + bve_e[...]
    ve_o = _dot(ee_, wve_o[...]) + bve_o[...]
    msg_out[0] = (vse + ve_e) * ex
    msg_out[1] = (vso + ve_o) * ex
    ex_out[...] = _dot(ex, c16[...])


def _upd_body(aggr, den, xn, x_in, st128,
              wih_ee, wih_oe, wih_eo, wih_oo, whh_e, whh_o, bg_e, bg_o,
              wsf_e, wsf_o, bsf_e, bsf_o, wo_e, wo_o, bo,
              g2, b2, wm1, bm1, wm2, bm2, gf, bf, out):
    den_e = _dot(den[0] + den[1], st128[...])
    rcp = 1.0 / (den_e + 1e-16)
    a_e = aggr[0] * rcp
    a_o = aggr[1] * rcp
    xn_ = xn[...]
    gate_e = jax.nn.sigmoid(_dot(a_e, wih_ee[...]) + _dot(a_o, wih_oe[...])
                            + _dot(xn_, whh_e[...]) + bg_e[...])
    gate_o = jax.nn.sigmoid(_dot(a_e, wih_eo[...]) + _dot(a_o, wih_oo[...])
                            + _dot(xn_, whh_o[...]) + bg_o[...])
    sf_e = _dot(xn_, wsf_e[...]) + bsf_e[...]
    sf_o = _dot(xn_, wsf_o[...]) + bsf_o[...]
    upd_e = a_e + gate_e * (sf_e - a_e)
    upd_o = a_o + gate_o * (sf_o - a_o)
    x = x_in[...] + _dot(upd_e, wo_e[...]) + _dot(upd_o, wo_o[...]) + bo[...]
    xn2 = _ln_in(x, g2[...], b2[...])
    h = jnp.maximum(_dot(xn2, wm1[...]) + bm1[...], 0.0)
    h = _dot(h, wm2[...]) + bm2[...]
    out[...] = _ln_in(x + h, gf[...], bf[...])


def _full(shape):
    nd = len(shape)
    return pl.BlockSpec(shape, lambda i: (0,) * nd)


_TC_PARAMS = pltpu.CompilerParams(
    dimension_semantics=("arbitrary",),
)


def _call_enc(fs, fd, *ws):
    ecnt = fs.shape[0]
    grid = (ecnt // BE,)
    especs = [pl.BlockSpec((BE, 128), lambda i: (i, 0))] * 2
    wspecs = [_full(w.shape) for w in ws]
    return pl.pallas_call(
        _enc_body,
        grid=grid,
        in_specs=especs + wspecs,
        out_specs=pl.BlockSpec((BE, D), lambda i: (i, 0)),
        out_shape=jax.ShapeDtypeStruct((ecnt, D), jnp.bfloat16),
        compiler_params=_TC_PARAMS,
    )(fs, fd, *ws)


def _call_prep(x, *ws):
    grid = (N // BN,)
    wspecs = [_full(w.shape) for w in ws]
    blk = pl.BlockSpec((BN, D), lambda i: (i, 0))
    hblk = pl.BlockSpec((BN, 128), lambda i: (i, 0))
    return pl.pallas_call(
        _prep_body,
        grid=grid,
        in_specs=[blk] + wspecs,
        out_specs=[blk] + [hblk] * 3,
        out_shape=[jax.ShapeDtypeStruct((N, D), jnp.float32)]
        + [jax.ShapeDtypeStruct((N, 128), jnp.int32)] * 3,
        compiler_params=_TC_PARAMS,
    )(x, *ws)


def _call_att(qd, ks, vs, ee, *ws):
    ecnt = qd.shape[0]
    grid = (ecnt // BE,)
    hblk = pl.BlockSpec((BE, 128), lambda i: (i, 0))
    eblk = pl.BlockSpec((BE, D), lambda i: (i, 0))
    wspecs = [_full(w.shape) for w in ws]
    return pl.pallas_call(
        _att_body,
        grid=grid,
        in_specs=[hblk] * 3 + [eblk] + wspecs,
        out_specs=[
            pl.BlockSpec((2, BE, 128), lambda i: (0, i, 0)),
            pl.BlockSpec((BE, 128), lambda i: (i, 0)),
        ],
        out_shape=[
            jax.ShapeDtypeStruct((2, ecnt, 128), jnp.float32),
            jax.ShapeDtypeStruct((ecnt, 128), jnp.float32),
        ],
        compiler_params=_TC_PARAMS,
    )(qd, ks, vs, ee, *ws)


def _call_upd(aggr, den, xn, x_in, *ws):
    grid = (N // BN,)
    nblk = pl.BlockSpec((BN, D), lambda i: (i, 0))
    ablk = pl.BlockSpec((2, BN, 128), lambda i: (0, i, 0))
    dblk = pl.BlockSpec((2, BN, 16), lambda i: (0, i, 0))
    wspecs = [_full(w.shape) for w in ws]
    return pl.pallas_call(
        _upd_body,
        grid=grid,
        in_specs=[ablk, dblk, nblk, nblk] + wspecs,
        out_specs=nblk,
        out_shape=jax.ShapeDtypeStruct((N, D), jnp.float32),
        compiler_params=_TC_PARAMS,
    )(aggr, den, xn, x_in, *ws)


def kernel(alg_embed, edge_index, positions, rotate_angles, rotate_mat, params):
    src = edge_index[0]
    dst = edge_index[1]

    tbl = jnp.concatenate(
        [positions[:, HS - 1, :],
         jnp.cos(rotate_angles)[:, None], jnp.sin(rotate_angles)[:, None],
         rotate_mat.reshape(N, 4), jnp.zeros((N, 120), jnp.float32)], axis=1)

    enc = params["edge_enc"]
    rel = params["rel"]
    agg = params["aggr"]

    def r2(v):
        return v.reshape(1, -1)

    z = jnp.zeros((128, 128), jnp.float32)
    sa_m = z.at[jnp.array([0, 1, 0, 1, 2, 3, 3, 2]),
                jnp.array([0, 2, 4, 6, 8, 9, 10, 11])].set(1.0)
    sa2_m = z.at[jnp.array([0, 1, 0, 1]),
                 jnp.array([1, 3, 5, 7])].set(1.0)
    sb_m = z.at[jnp.array([4, 4, 6, 6, 5, 5, 7, 7, 2, 3, 2, 3]),
                jnp.array([0, 1, 2, 3, 4, 5, 6, 7, 8, 9, 10, 11])].set(1.0)
    scm = z.at[jnp.array([0, 2, 1, 3, 4, 6, 5, 7, 8, 9, 10, 11]),
               jnp.array([0, 0, 0, 0, 1, 1, 1, 1, 2, 2, 3, 3])].set(
        jnp.array([1., 1., -1., -1., 1., 1., -1., -1., 1., 1., 1., -1.]))
    w1e_ext = jnp.zeros((128, D), jnp.float32).at[0:2].set(enc["l1"]["w"])
    w1r_ext = jnp.zeros((128, D), jnp.float32).at[2:4].set(rel["l1"]["w"])
    we128 = scm @ w1e_ext
    wr128 = scm @ w1r_ext

    enc_ws = (
        sa_m, sa2_m, sb_m,
        we128, r2(enc["l1"]["b"]), r2(enc["ln"]["g"]), r2(enc["ln"]["b"]),
        enc["l2"]["w"].astype(jnp.bfloat16), r2(enc["l2"]["b"]),
        wr128, r2(rel["l1"]["b"]), r2(rel["ln"]["g"]), r2(rel["ln"]["b"]),
        rel["l2"]["w"].astype(jnp.bfloat16), r2(rel["l2"]["b"]),
        r2(agg["ln1"]["g"]), r2(agg["ln1"]["b"]),
        agg["lin"]["w"].astype(jnp.bfloat16), r2(agg["lin"]["b"]),
        r2(agg["ln2"]["g"]), r2(agg["ln2"]["b"]),
    )

    lp = params["layers"][-1]

    def cols(w, p):
        return w[:, p::2]

    def colb(b, p):
        return b[p::2].reshape(1, -1)

    def colsb(w, p):
        return w[:, p::2].astype(jnp.bfloat16)

    xn, q, kn, vn = _call_prep(
        alg_embed,
        cols(lp["q"]["w"], 0), cols(lp["q"]["w"], 1),
        colb(lp["q"]["b"], 0), colb(lp["q"]["b"], 1),
        cols(lp["kn"]["w"], 0), cols(lp["kn"]["w"], 1),
        colb(lp["kn"]["b"], 0), colb(lp["kn"]["b"], 1),
        cols(lp["vn"]["w"], 0), cols(lp["vn"]["w"], 1),
        colb(lp["vn"]["b"], 0), colb(lp["vn"]["b"], 1),
        r2(lp["norm1"]["g"]), r2(lp["norm1"]["b"]),
    )

    m16 = jnp.kron(jnp.eye(H, dtype=jnp.float32), jnp.ones((16, 16), jnp.float32))
    c16 = jnp.concatenate(
        [jnp.kron(jnp.eye(H, dtype=jnp.float32),
                  jnp.full((16, 1), 1.0 / 16, jnp.float32)),
         jnp.zeros((128, 120), jnp.float32)], axis=1)
    st128 = jnp.concatenate(
        [jnp.kron(jnp.eye(H, dtype=jnp.float32), jnp.ones((1, 16), jnp.float32)),
         jnp.zeros((8, 128), jnp.float32)], axis=0)

    att_ws = (
        colsb(lp["ke"]["w"], 0), colsb(lp["ke"]["w"], 1),
        colb(lp["ke"]["b"], 0), colb(lp["ke"]["b"], 1),
        colsb(lp["ve"]["w"], 0), colsb(lp["ve"]["w"], 1),
        colb(lp["ve"]["b"], 0), colb(lp["ve"]["b"], 1),
        m16, c16,
    )

    zn = jnp.zeros((N, 128), jnp.float32)
    zd = jnp.zeros((N, 16), jnp.float32)

    fs, fd = _gather_feats_f(tbl, src, dst)
    ee = _call_enc(fs, fd, *enc_ws)
    qd, ks, vs = _gather_qkv_f(q, kn, vn, src, dst)
    msg, ex = _call_att(qd, ks, vs, ee, *att_ws)
    aggr = _scatter_f(dst, msg, zn)
    den = _scatter_den_f(dst, ex, zd)

    def rows_(w, p):
        return w[p::2, :]

    bg = lp["ih"]["b"] + lp["hh"]["b"]
    out = _call_upd(
        aggr, den, xn, alg_embed, st128,
        rows_(cols(lp["ih"]["w"], 0), 0), rows_(cols(lp["ih"]["w"], 0), 1),
        rows_(cols(lp["ih"]["w"], 1), 0), rows_(cols(lp["ih"]["w"], 1), 1),
        cols(lp["hh"]["w"], 0), cols(lp["hh"]["w"], 1),
        colb(bg, 0), colb(bg, 1),
        cols(lp["self"]["w"], 0), cols(lp["self"]["w"], 1),
        colb(lp["self"]["b"], 0), colb(lp["self"]["b"], 1),
        rows_(lp["out"]["w"], 0), rows_(lp["out"]["w"], 1), r2(lp["out"]["b"]),
        r2(lp["norm2"]["g"]), r2(lp["norm2"]["b"]),
        lp["mlp1"]["w"], r2(lp["mlp1"]["b"]), lp["mlp2"]["w"], r2(lp["mlp2"]["b"]),
        r2(params["norm"]["g"]), r2(params["norm"]["b"]),
    )
    return out
